# baseline (speedup 1.0000x reference)
import functools
import math

import jax
import jax.numpy as jnp
from jax import lax
from jax.experimental import pallas as pl
from jax.experimental.pallas import tpu as pltpu
from jax.experimental.pallas import tpu_sc as plsc

D_MODEL = 768
_SCALE = math.sqrt(D_MODEL)
_LANES = 16
_VECS = D_MODEL // _LANES

_NUM_CORES = 2
_NUM_SUBCORES = 16
_NW = _NUM_CORES * _NUM_SUBCORES

_CHUNK = 32


@functools.cache
def _build(B: int, V: int):
    assert B % _NW == 0
    bpw = B // _NW
    assert bpw % _CHUNK == 0
    nchunk = bpw // _CHUNK

    mesh = plsc.VectorSubcoreMesh(
        core_axis_name="c", subcore_axis_name="s",
        num_cores=_NUM_CORES, num_subcores=_NUM_SUBCORES)

    @functools.partial(
        pl.kernel,
        mesh=mesh,
        out_type=jax.ShapeDtypeStruct((B, D_MODEL), jnp.float32),
        scratch_types=[
            pltpu.VMEM((bpw,), jnp.int32),
            pltpu.VMEM((2, _CHUNK, D_MODEL), jnp.float32),
            pltpu.VMEM((2, _CHUNK, D_MODEL), jnp.float32),
            pltpu.SemaphoreType.DMA,
            pltpu.SemaphoreType.DMA,
        ],
    )
    def emb_kernel(x_hbm, lut_hbm, out_hbm, idx_v, gbufs, wbufs, gsem, wsem):
        wid = lax.axis_index("s") * _NUM_CORES + lax.axis_index("c")
        base = wid * bpw
        pltpu.sync_copy(x_hbm.at[pl.ds(base, bpw)], idx_v)

        def start_gather(i):
            return pltpu.async_copy(
                lut_hbm.at[idx_v.at[pl.ds(i * _CHUNK, _CHUNK)]],
                gbufs.at[i % 2], gsem)

        def start_write(i):
            return pltpu.async_copy(
                wbufs.at[i % 2],
                out_hbm.at[pl.ds(base + i * _CHUNK, _CHUNK)], wsem)

        def scale(slot):
            def row_body(r, carry):
                for c in range(_VECS):
                    sl = pl.ds(c * _LANES, _LANES)
                    wbufs[slot, r, sl] = gbufs[slot, r, sl] * _SCALE
                return carry
            lax.fori_loop(0, _CHUNK, row_body, 0)

        gh = [None] * nchunk
        wh = [None] * nchunk
        gh[0] = start_gather(0)
        if nchunk > 1:
            gh[1] = start_gather(1)
        for i in range(nchunk):
            gh[i].wait()
            if i >= 2:
                wh[i - 2].wait()
            scale(i % 2)
            wh[i] = start_write(i)
            if i + 2 < nchunk:
                gh[i + 2] = start_gather(i + 2)
        for i in range(max(0, nchunk - 2), nchunk):
            wh[i].wait()

    return emb_kernel


def kernel(x, lut):
    B = x.shape[0] * x.shape[1]
    out = _build(B, lut.shape[0])(x.reshape(-1).astype(jnp.int32), lut)
    return out.reshape(*x.shape, D_MODEL)


# speedup vs baseline: 1.8071x; 1.8071x over previous
import functools
import math

import jax
import jax.numpy as jnp
from jax import lax
from jax.experimental import pallas as pl
from jax.experimental.pallas import tpu as pltpu
from jax.experimental.pallas import tpu_sc as plsc

D_MODEL = 768
_SCALE = math.sqrt(D_MODEL)
_LANES = 16
_VECS = D_MODEL // _LANES

_NUM_CORES = 2
_NUM_SUBCORES = 16
_NW = _NUM_CORES * _NUM_SUBCORES

_CHUNK = 64


@functools.cache
def _build(B: int, V: int):
    assert B % _NW == 0
    bpw = B // _NW
    assert bpw % _CHUNK == 0
    nchunk = bpw // _CHUNK

    mesh = plsc.VectorSubcoreMesh(
        core_axis_name="c", subcore_axis_name="s",
        num_cores=_NUM_CORES, num_subcores=_NUM_SUBCORES)

    @functools.partial(
        pl.kernel,
        mesh=mesh,
        out_type=jax.ShapeDtypeStruct((B, D_MODEL), jnp.float32),
        scratch_types=[
            pltpu.VMEM((bpw,), jnp.int32),
            pltpu.VMEM((2, _CHUNK, D_MODEL), jnp.float32),
            pltpu.SemaphoreType.DMA,
            pltpu.SemaphoreType.DMA,
        ],
    )
    def emb_kernel(x_hbm, lut_hbm, out_hbm, idx_v, bufs, gsem, wsem):
        wid = lax.axis_index("s") * _NUM_CORES + lax.axis_index("c")
        base = wid * bpw
        pltpu.sync_copy(x_hbm.at[pl.ds(base, bpw)], idx_v)

        def start_gather(i):
            return pltpu.async_copy(
                lut_hbm.at[idx_v.at[pl.ds(i * _CHUNK, _CHUNK)]],
                bufs.at[i % 2], gsem)

        def start_write(i):
            return pltpu.async_copy(
                bufs.at[i % 2],
                out_hbm.at[pl.ds(base + i * _CHUNK, _CHUNK)], wsem)

        def scale(slot):
            def row_body(r, carry):
                for c in range(_VECS):
                    sl = pl.ds(c * _LANES, _LANES)
                    bufs[slot, r, sl] = bufs[slot, r, sl] * _SCALE
                return carry
            lax.fori_loop(0, _CHUNK, row_body, 0)

        gh = [None] * nchunk
        wh = [None] * nchunk
        gh[0] = start_gather(0)
        for i in range(nchunk):
            if i + 1 < nchunk:
                if i >= 1:
                    wh[i - 1].wait()
                gh[i + 1] = start_gather(i + 1)
            gh[i].wait()
            scale(i % 2)
            wh[i] = start_write(i)
        for i in range(max(0, nchunk - 2), nchunk):
            wh[i].wait()

    return emb_kernel


def kernel(x, lut):
    B = x.shape[0] * x.shape[1]
    out = _build(B, lut.shape[0])(x.reshape(-1).astype(jnp.int32), lut)
    return out.reshape(*x.shape, D_MODEL)
